# baseline (speedup 1.0000x reference)
import functools

import jax
import jax.numpy as jnp
from jax import lax
from jax.experimental import pallas as pl
from jax.experimental.pallas import tpu as pltpu
from jax.experimental.pallas import tpu_sc as plsc

E = 8
K = 2
T = 2048
H = 1024
F = 768
H2 = H // 2
BLK = 256
P = T * K + E * BLK
NB = P // BLK
BT = 256
NBR = T // BT

NC, NS, L = 2, 16, 16
NW = NC * NS
PAIRS = T * K

_SC_PARAMS = pltpu.CompilerParams(needs_layout_passes=False)

_mesh = functools.partial(
    plsc.VectorSubcoreMesh, core_axis_name="c", subcore_axis_name="s")

_MASK_HI = -65536


def _pack_rows(y):
    lo = lax.bitcast_convert_type(
        y[:, :H2].astype(jnp.bfloat16).astype(jnp.float32), jnp.int32)
    hi = lax.bitcast_convert_type(
        y[:, H2:].astype(jnp.bfloat16).astype(jnp.float32), jnp.int32)
    return jnp.bitwise_or(lax.shift_right_logical(lo, 16),
                          jnp.bitwise_and(hi, _MASK_HI))


def _unpack_rows(xi):
    lo = lax.bitcast_convert_type(lax.shift_left(xi, 16), jnp.float32)
    hi = lax.bitcast_convert_type(jnp.bitwise_and(xi, _MASK_HI), jnp.float32)
    return jnp.concatenate([lo, hi], axis=1)


def _top2(x, gw):
    logits = lax.dot_general(x, gw, (((1,), (1,)), ((), ())),
                             preferred_element_type=jnp.float32)
    s = jax.nn.softmax(logits, axis=-1)
    lanes = lax.broadcasted_iota(jnp.int32, s.shape, 1)
    m1 = jnp.max(s, axis=-1, keepdims=True)
    i1 = jnp.argmax(s, axis=-1)[:, None]
    s2 = jnp.where(lanes == i1, -jnp.inf, s)
    m2 = jnp.max(s2, axis=-1, keepdims=True)
    i2 = jnp.argmax(s2, axis=-1)[:, None]
    denom = m1 + m2
    oh1 = (lanes == i1).astype(jnp.float32)
    oh2 = (lanes == i2).astype(jnp.float32)
    return oh1, oh2, m1 / denom, m2 / denom


def _router_body(x_ref, gw_ref, pos_ref, wts_ref, xi_ref, bexp_ref, nrows_ref,
                 xmap_ref, cnt_ref):
    p = pl.program_id(0)
    i = pl.program_id(1)
    x = x_ref[...]
    oh1, oh2, w1, w2 = _top2(x, gw_ref[...])
    ohsum = oh1 + oh2

    @pl.when(p == 0)
    def _pass0():
        hist = jnp.sum(ohsum, axis=0, keepdims=True)
        rows = lax.broadcasted_iota(jnp.int32, (NBR, E), 0)
        cnt_ref[...] = jnp.where(rows == i, hist, cnt_ref[...])

    @pl.when(p == 1)
    def _pass1():
        xi_ref[...] = _pack_rows(x)
        cnt = cnt_ref[...]
        ones_row = jnp.ones((1, NBR), jnp.float32)
        counts = lax.dot_general(ones_row, cnt, (((1,), (0,)), ((), ())),
                                 preferred_element_type=jnp.float32)
        sel = (lax.broadcasted_iota(jnp.int32, (1, NBR), 1) < i
               ).astype(jnp.float32)
        prefix = lax.dot_general(sel, cnt, (((1,), (0,)), ((), ())),
                                 preferred_element_type=jnp.float32)
        nblk = jnp.floor((counts + (BLK - 1)) * (1.0 / BLK))
        tri_e = (lax.broadcasted_iota(jnp.int32, (E, E), 0)
                 < lax.broadcasted_iota(jnp.int32, (E, E), 1)
                 ).astype(jnp.float32)
        start = lax.dot_general(nblk, tri_e, (((1,), (0,)), ((), ())),
                                preferred_element_type=jnp.float32)
        pad_off = start * BLK

        tl = (lax.broadcasted_iota(jnp.int32, (BT, BT), 1)
              < lax.broadcasted_iota(jnp.int32, (BT, BT), 0)
              ).astype(jnp.float32)
        pre = lax.dot_general(tl, ohsum, (((1,), (0,)), ((), ())),
                              preferred_element_type=jnp.float32)
        base = pad_off + prefix
        pos1 = jnp.sum((pre + base) * oh1, axis=1, keepdims=True)
        pos2 = jnp.sum((pre + base) * oh2, axis=1, keepdims=True)
        pos_ref[...] = jnp.concatenate([pos1, pos2], axis=1).astype(jnp.int32)
        wts_ref[...] = jnp.concatenate([w1, w2], axis=1)

        @pl.when(i == 0)
        def _meta():
            occ = start[:, E - 1:E] + nblk[:, E - 1:E]
            nbs = lax.broadcasted_iota(jnp.int32, (1, NB), 1
                                       ).astype(jnp.float32)
            nbv = jnp.minimum(nbs, occ - 1.0)
            bexp = -jnp.ones((1, NB), jnp.float32)
            csel = jnp.zeros((1, NB), jnp.float32)
            psel = jnp.zeros((1, NB), jnp.float32)
            for e in range(E):
                st_e = start[:, e:e + 1]
                bexp = bexp + (st_e <= nbv).astype(jnp.float32)
            for e in range(E):
                is_e = (bexp == e).astype(jnp.float32)
                csel = csel + is_e * counts[:, e:e + 1]
                psel = psel + is_e * pad_off[:, e:e + 1]
            nrows = jnp.clip(csel - (nbv * BLK - psel), 0.0, float(BLK))
            nrows = jnp.where(nbs < occ, nrows, 0.0)
            bexp_ref[...] = bexp.astype(jnp.int32)
            nrows_ref[...] = nrows.astype(jnp.int32)
            xmap_ref[...] = nbv.astype(jnp.int32)


def _run_router(x, gate_weight):
    return pl.pallas_call(
        _router_body,
        grid=(2, NBR),
        in_specs=[
            pl.BlockSpec((BT, H), lambda p, i: (i, 0)),
            pl.BlockSpec((E, H), lambda p, i: (0, 0)),
        ],
        out_specs=[
            pl.BlockSpec((BT, K), lambda p, i: (i, 0)),
            pl.BlockSpec((BT, K), lambda p, i: (i, 0)),
            pl.BlockSpec((BT, H2), lambda p, i: (i, 0)),
            pl.BlockSpec((1, NB), lambda p, i: (0, 0)),
            pl.BlockSpec((1, NB), lambda p, i: (0, 0)),
            pl.BlockSpec((1, NB), lambda p, i: (0, 0)),
        ],
        out_shape=[
            jax.ShapeDtypeStruct((T, K), jnp.int32),
            jax.ShapeDtypeStruct((T, K), jnp.float32),
            jax.ShapeDtypeStruct((T, H2), jnp.int32),
            jax.ShapeDtypeStruct((1, NB), jnp.int32),
            jax.ShapeDtypeStruct((1, NB), jnp.int32),
            jax.ShapeDtypeStruct((1, NB), jnp.int32),
        ],
        scratch_shapes=[pltpu.VMEM((NBR, E), jnp.float32)],
        compiler_params=pltpu.CompilerParams(
            dimension_semantics=("arbitrary", "arbitrary")),
    )(x, gate_weight)


_RPW = P // NW


def _dispatch_body(xi_hbm, pos_hbm, w_hbm, xs_hbm, wsort_hbm,
                   posv, wv, tokv, pairv, wsv, rowsv, sem):
    cid = lax.axis_index("c")
    sid = lax.axis_index("s")
    wid = sid * NC + cid
    base = wid * _RPW

    pltpu.sync_copy(pos_hbm, posv)
    pltpu.sync_copy(w_hbm, wv)
    for c in range(_RPW // L):
        tokv[pl.ds(c * L, L)] = jnp.zeros((L,), jnp.int32)
        pairv[pl.ds(c * L, L)] = jnp.zeros((L,), jnp.int32)

    def _scan(c, _):
        pp = posv[pl.ds(c * L, L)]
        rel = pp - base
        mask = jnp.logical_and(rel >= 0, rel < _RPW)
        rel = jnp.clip(rel, 0, _RPW - 1)
        pair = c * L + lax.iota(jnp.int32, L)
        plsc.store_scatter(pairv, [rel], pair, mask=mask)
        plsc.store_scatter(tokv, [rel],
                           lax.shift_right_logical(pair, 1), mask=mask)
        return 0

    lax.fori_loop(0, PAIRS // L, _scan, 0)

    def _wsel(c, _):
        pair = pairv[pl.ds(c * L, L)]
        wsv[pl.ds(c * L, L)] = plsc.load_gather(wv, [pair])
        return 0

    lax.fori_loop(0, _RPW // L, _wsel, 0)
    pltpu.sync_copy(wsv, wsort_hbm.at[pl.ds(base, _RPW)])
    pltpu.async_copy(xi_hbm.at[tokv], rowsv, sem).wait()
    pltpu.sync_copy(rowsv, xs_hbm.at[pl.ds(base, _RPW)])


def _run_dispatch(xi, pos_flat, w_flat):
    return pl.kernel(
        _dispatch_body,
        out_type=[
            jax.ShapeDtypeStruct((P, H2), jnp.int32),
            jax.ShapeDtypeStruct((P,), jnp.float32),
        ],
        mesh=_mesh(),
        scratch_types=[
            pltpu.VMEM((PAIRS,), jnp.int32),
            pltpu.VMEM((PAIRS,), jnp.float32),
            pltpu.VMEM((_RPW,), jnp.int32),
            pltpu.VMEM((_RPW,), jnp.int32),
            pltpu.VMEM((_RPW,), jnp.float32),
            pltpu.VMEM((_RPW, H2), jnp.int32),
            pltpu.SemaphoreType.DMA,
        ],
        compiler_params=_SC_PARAMS,
    )(xi, pos_flat, w_flat)


def _gemm_body(bexp_ref, nrows_ref, xmap_ref, xs_ref, wcol_ref,
               gw_ref, uw_ref, dw_ref, out_ref):
    i = pl.program_id(0)

    @pl.when(nrows_ref[i] > 0)
    def _():
        x = _unpack_rows(xs_ref[...])
        g = lax.dot_general(x, gw_ref[0], (((1,), (0,)), ((), ())),
                            preferred_element_type=jnp.float32)
        u = lax.dot_general(x, uw_ref[0], (((1,), (0,)), ((), ())),
                            preferred_element_type=jnp.float32)
        hh = g * jax.nn.sigmoid(g) * u
        d = lax.dot_general(hh, dw_ref[0], (((1,), (0,)), ((), ())),
                            preferred_element_type=jnp.float32)
        out_ref[...] = _pack_rows(wcol_ref[...] * d)


def _run_gemm(bexp, nrows, xmap, xs, wcol, gw, uw, dw):
    grid_spec = pltpu.PrefetchScalarGridSpec(
        num_scalar_prefetch=3,
        grid=(NB,),
        in_specs=[
            pl.BlockSpec((BLK, H2), lambda i, be, nr, xm: (xm[i], 0)),
            pl.BlockSpec((BLK, 1), lambda i, be, nr, xm: (xm[i], 0)),
            pl.BlockSpec((1, H, F), lambda i, be, nr, xm: (be[i], 0, 0)),
            pl.BlockSpec((1, H, F), lambda i, be, nr, xm: (be[i], 0, 0)),
            pl.BlockSpec((1, F, H), lambda i, be, nr, xm: (be[i], 0, 0)),
        ],
        out_specs=pl.BlockSpec((BLK, H2), lambda i, be, nr, xm: (xm[i], 0)),
    )
    return pl.pallas_call(
        _gemm_body,
        grid_spec=grid_spec,
        out_shape=jax.ShapeDtypeStruct((P, H2), jnp.int32),
        compiler_params=pltpu.CompilerParams(
            dimension_semantics=("arbitrary",)),
    )(bexp, nrows, xmap, xs, wcol, gw, uw, dw)


_TPW = T // NW
_THALF = _TPW // 2


def _combine_body(ds_hbm, pos_hbm, out_hbm, idxv, rowsv, outv, sem):
    cid = lax.axis_index("c")
    sid = lax.axis_index("s")
    wid = sid * NC + cid
    tbase = wid * _TPW
    pltpu.sync_copy(pos_hbm.at[pl.ds(tbase * K, _TPW * K)], idxv)
    pltpu.async_copy(ds_hbm.at[idxv], rowsv, sem).wait()

    for half in range(2):
        def _addtok(tt, _):
            r = (half * _THALF + tt) * 2
            for v in range(H2 // L):
                sl = pl.ds(v * L, L)
                wa = rowsv[r, sl]
                wb = rowsv[r + 1, sl]
                lo = (plsc.bitcast(lax.shift_left(wa, 16), jnp.float32)
                      + plsc.bitcast(lax.shift_left(wb, 16), jnp.float32))
                hi = (plsc.bitcast(jnp.bitwise_and(wa, _MASK_HI), jnp.float32)
                      + plsc.bitcast(jnp.bitwise_and(wb, _MASK_HI),
                                     jnp.float32))
                outv[tt, sl] = lo
                outv[tt, pl.ds(H2 + v * L, L)] = hi
            return 0

        lax.fori_loop(0, _THALF, _addtok, 0)
        pltpu.sync_copy(
            outv, out_hbm.at[pl.ds(tbase + half * _THALF, _THALF)])


def _run_combine(ds, pos_flat):
    return pl.kernel(
        _combine_body,
        out_type=jax.ShapeDtypeStruct((T, H), jnp.float32),
        mesh=_mesh(),
        scratch_types=[
            pltpu.VMEM((_TPW * K,), jnp.int32),
            pltpu.VMEM((_TPW * K, H2), jnp.int32),
            pltpu.VMEM((_THALF, H), jnp.float32),
            pltpu.SemaphoreType.DMA,
        ],
        compiler_params=_SC_PARAMS,
    )(ds, pos_flat)


def kernel(hidden_states, gate_weight, gate_proj_w, up_proj_w, down_proj_w):
    b, s, h = hidden_states.shape
    x = hidden_states.reshape(-1, h)

    pos, wts, xi, bexp, nrows, xmap = _run_router(x, gate_weight)
    pos_flat = pos.reshape(-1)
    xs, wsort = _run_dispatch(xi, pos_flat, wts.reshape(-1))
    ds = _run_gemm(bexp.reshape(-1), nrows.reshape(-1), xmap.reshape(-1),
                   xs, wsort.reshape(P, 1),
                   gate_proj_w, up_proj_w, down_proj_w)
    out = _run_combine(ds, pos_flat)
    return out.reshape(b, s, h)


# speedup vs baseline: 1.0011x; 1.0011x over previous
import functools

import jax
import jax.numpy as jnp
from jax import lax
from jax.experimental import pallas as pl
from jax.experimental.pallas import tpu as pltpu
from jax.experimental.pallas import tpu_sc as plsc

E = 8
K = 2
T = 2048
H = 1024
F = 768
H2 = H // 2
BLK = 256
P = T * K + E * BLK
NB = P // BLK
BT = 256
NBR = T // BT

NC, NS, L = 2, 16, 16
NW = NC * NS
PAIRS = T * K

_SC_PARAMS = pltpu.CompilerParams(needs_layout_passes=False)

_mesh = functools.partial(
    plsc.VectorSubcoreMesh, core_axis_name="c", subcore_axis_name="s")

_MASK_HI = -65536


def _pack_rows(y):
    lo = lax.bitcast_convert_type(
        y[:, :H2].astype(jnp.bfloat16).astype(jnp.float32), jnp.int32)
    hi = lax.bitcast_convert_type(
        y[:, H2:].astype(jnp.bfloat16).astype(jnp.float32), jnp.int32)
    return jnp.bitwise_or(lax.shift_right_logical(lo, 16),
                          jnp.bitwise_and(hi, _MASK_HI))


def _unpack_rows(xi):
    lo = lax.bitcast_convert_type(lax.shift_left(xi, 16), jnp.float32)
    hi = lax.bitcast_convert_type(jnp.bitwise_and(xi, _MASK_HI), jnp.float32)
    return jnp.concatenate([lo, hi], axis=1)


def _top2(x, gw):
    logits = lax.dot_general(x, gw, (((1,), (1,)), ((), ())),
                             preferred_element_type=jnp.float32)
    s = jax.nn.softmax(logits, axis=-1)
    lanes = lax.broadcasted_iota(jnp.int32, s.shape, 1)
    m1 = jnp.max(s, axis=-1, keepdims=True)
    i1 = jnp.argmax(s, axis=-1)[:, None]
    s2 = jnp.where(lanes == i1, -jnp.inf, s)
    m2 = jnp.max(s2, axis=-1, keepdims=True)
    i2 = jnp.argmax(s2, axis=-1)[:, None]
    denom = m1 + m2
    oh1 = (lanes == i1).astype(jnp.float32)
    oh2 = (lanes == i2).astype(jnp.float32)
    return oh1, oh2, m1 / denom, m2 / denom


def _router_body(x_ref, gw_ref, pos_ref, wts_ref, xi_ref, bexp_ref, nrows_ref,
                 xmap_ref, cnt_ref):
    p = pl.program_id(0)
    i = pl.program_id(1)
    x = x_ref[...]
    oh1, oh2, w1, w2 = _top2(x, gw_ref[...])
    ohsum = oh1 + oh2

    @pl.when(p == 0)
    def _pass0():
        hist = jnp.sum(ohsum, axis=0, keepdims=True)
        rows = lax.broadcasted_iota(jnp.int32, (NBR, E), 0)
        cnt_ref[...] = jnp.where(rows == i, hist, cnt_ref[...])

    @pl.when(p == 1)
    def _pass1():
        xi_ref[...] = _pack_rows(x)
        cnt = cnt_ref[...]
        ones_row = jnp.ones((1, NBR), jnp.float32)
        counts = lax.dot_general(ones_row, cnt, (((1,), (0,)), ((), ())),
                                 preferred_element_type=jnp.float32)
        sel = (lax.broadcasted_iota(jnp.int32, (1, NBR), 1) < i
               ).astype(jnp.float32)
        prefix = lax.dot_general(sel, cnt, (((1,), (0,)), ((), ())),
                                 preferred_element_type=jnp.float32)
        nblk = jnp.floor((counts + (BLK - 1)) * (1.0 / BLK))
        tri_e = (lax.broadcasted_iota(jnp.int32, (E, E), 0)
                 < lax.broadcasted_iota(jnp.int32, (E, E), 1)
                 ).astype(jnp.float32)
        start = lax.dot_general(nblk, tri_e, (((1,), (0,)), ((), ())),
                                preferred_element_type=jnp.float32)
        pad_off = start * BLK

        tl = (lax.broadcasted_iota(jnp.int32, (BT, BT), 1)
              < lax.broadcasted_iota(jnp.int32, (BT, BT), 0)
              ).astype(jnp.float32)
        pre = lax.dot_general(tl, ohsum, (((1,), (0,)), ((), ())),
                              preferred_element_type=jnp.float32)
        base = pad_off + prefix
        pos1 = jnp.sum((pre + base) * oh1, axis=1, keepdims=True)
        pos2 = jnp.sum((pre + base) * oh2, axis=1, keepdims=True)
        pos_ref[...] = jnp.concatenate([pos1, pos2], axis=1).astype(jnp.int32)
        wts_ref[...] = jnp.concatenate([w1, w2], axis=1)

        @pl.when(i == 0)
        def _meta():
            occ = start[:, E - 1:E] + nblk[:, E - 1:E]
            nbs = lax.broadcasted_iota(jnp.int32, (1, NB), 1
                                       ).astype(jnp.float32)
            nbv = jnp.minimum(nbs, occ - 1.0)
            bexp = -jnp.ones((1, NB), jnp.float32)
            csel = jnp.zeros((1, NB), jnp.float32)
            psel = jnp.zeros((1, NB), jnp.float32)
            for e in range(E):
                st_e = start[:, e:e + 1]
                bexp = bexp + (st_e <= nbv).astype(jnp.float32)
            for e in range(E):
                is_e = (bexp == e).astype(jnp.float32)
                csel = csel + is_e * counts[:, e:e + 1]
                psel = psel + is_e * pad_off[:, e:e + 1]
            nrows = jnp.clip(csel - (nbv * BLK - psel), 0.0, float(BLK))
            nrows = jnp.where(nbs < occ, nrows, 0.0)
            bexp_ref[...] = bexp.astype(jnp.int32)
            nrows_ref[...] = nrows.astype(jnp.int32)
            xmap_ref[...] = nbv.astype(jnp.int32)


def _run_router(x, gate_weight):
    return pl.pallas_call(
        _router_body,
        grid=(2, NBR),
        in_specs=[
            pl.BlockSpec((BT, H), lambda p, i: (i, 0)),
            pl.BlockSpec((E, H), lambda p, i: (0, 0)),
        ],
        out_specs=[
            pl.BlockSpec((BT, K), lambda p, i: (i, 0)),
            pl.BlockSpec((BT, K), lambda p, i: (i, 0)),
            pl.BlockSpec((BT, H2), lambda p, i: (i, 0)),
            pl.BlockSpec((1, NB), lambda p, i: (0, 0)),
            pl.BlockSpec((1, NB), lambda p, i: (0, 0)),
            pl.BlockSpec((1, NB), lambda p, i: (0, 0)),
        ],
        out_shape=[
            jax.ShapeDtypeStruct((T, K), jnp.int32),
            jax.ShapeDtypeStruct((T, K), jnp.float32),
            jax.ShapeDtypeStruct((T, H2), jnp.int32),
            jax.ShapeDtypeStruct((1, NB), jnp.int32),
            jax.ShapeDtypeStruct((1, NB), jnp.int32),
            jax.ShapeDtypeStruct((1, NB), jnp.int32),
        ],
        scratch_shapes=[pltpu.VMEM((NBR, E), jnp.float32)],
        compiler_params=pltpu.CompilerParams(
            dimension_semantics=("arbitrary", "arbitrary")),
    )(x, gate_weight)


_RPW = P // NW


def _dispatch_body(xi_hbm, pos_hbm, w_hbm, xs_hbm, wsort_hbm,
                   posv, wv, tokv, pairv, wsv, rowsv, sem):
    cid = lax.axis_index("c")
    sid = lax.axis_index("s")
    wid = sid * NC + cid
    base = wid * _RPW

    pltpu.sync_copy(pos_hbm, posv)
    pltpu.sync_copy(w_hbm, wv)
    for c in range(_RPW // L):
        tokv[pl.ds(c * L, L)] = jnp.zeros((L,), jnp.int32)
        pairv[pl.ds(c * L, L)] = jnp.zeros((L,), jnp.int32)

    def _scan(c, _):
        pp = posv[pl.ds(c * L, L)]
        rel = pp - base
        mask = jnp.logical_and(rel >= 0, rel < _RPW)
        rel = jnp.clip(rel, 0, _RPW - 1)
        pair = c * L + lax.iota(jnp.int32, L)
        plsc.store_scatter(pairv, [rel], pair, mask=mask)
        plsc.store_scatter(tokv, [rel],
                           lax.shift_right_logical(pair, 1), mask=mask)
        return 0

    lax.fori_loop(0, PAIRS // L, _scan, 0)

    def _wsel(c, _):
        pair = pairv[pl.ds(c * L, L)]
        wsv[pl.ds(c * L, L)] = plsc.load_gather(wv, [pair])
        return 0

    lax.fori_loop(0, _RPW // L, _wsel, 0)
    pltpu.sync_copy(wsv, wsort_hbm.at[pl.ds(base, _RPW)])
    half = _RPW // 2
    d0 = pltpu.async_copy(
        xi_hbm.at[tokv.at[pl.ds(0, half)]], rowsv.at[pl.ds(0, half)], sem)
    d1 = pltpu.async_copy(
        xi_hbm.at[tokv.at[pl.ds(half, half)]], rowsv.at[pl.ds(half, half)],
        sem)
    d0.wait()
    d1.wait()
    pltpu.sync_copy(rowsv, xs_hbm.at[pl.ds(base, _RPW)])


def _run_dispatch(xi, pos_flat, w_flat):
    return pl.kernel(
        _dispatch_body,
        out_type=[
            jax.ShapeDtypeStruct((P, H2), jnp.int32),
            jax.ShapeDtypeStruct((P,), jnp.float32),
        ],
        mesh=_mesh(),
        scratch_types=[
            pltpu.VMEM((PAIRS,), jnp.int32),
            pltpu.VMEM((PAIRS,), jnp.float32),
            pltpu.VMEM((_RPW,), jnp.int32),
            pltpu.VMEM((_RPW,), jnp.int32),
            pltpu.VMEM((_RPW,), jnp.float32),
            pltpu.VMEM((_RPW, H2), jnp.int32),
            pltpu.SemaphoreType.DMA,
        ],
        compiler_params=_SC_PARAMS,
    )(xi, pos_flat, w_flat)


def _gemm_body(bexp_ref, nrows_ref, xmap_ref, xs_ref, wcol_ref,
               gw_ref, uw_ref, dw_ref, out_ref):
    i = pl.program_id(0)

    @pl.when(nrows_ref[i] > 0)
    def _():
        x = _unpack_rows(xs_ref[...])
        g = lax.dot_general(x, gw_ref[0], (((1,), (0,)), ((), ())),
                            preferred_element_type=jnp.float32)
        u = lax.dot_general(x, uw_ref[0], (((1,), (0,)), ((), ())),
                            preferred_element_type=jnp.float32)
        hh = g * jax.nn.sigmoid(g) * u
        d = lax.dot_general(hh, dw_ref[0], (((1,), (0,)), ((), ())),
                            preferred_element_type=jnp.float32)
        out_ref[...] = _pack_rows(wcol_ref[...] * d)


def _run_gemm(bexp, nrows, xmap, xs, wcol, gw, uw, dw):
    grid_spec = pltpu.PrefetchScalarGridSpec(
        num_scalar_prefetch=3,
        grid=(NB,),
        in_specs=[
            pl.BlockSpec((BLK, H2), lambda i, be, nr, xm: (xm[i], 0)),
            pl.BlockSpec((BLK, 1), lambda i, be, nr, xm: (xm[i], 0)),
            pl.BlockSpec((1, H, F), lambda i, be, nr, xm: (be[i], 0, 0)),
            pl.BlockSpec((1, H, F), lambda i, be, nr, xm: (be[i], 0, 0)),
            pl.BlockSpec((1, F, H), lambda i, be, nr, xm: (be[i], 0, 0)),
        ],
        out_specs=pl.BlockSpec((BLK, H2), lambda i, be, nr, xm: (xm[i], 0)),
    )
    return pl.pallas_call(
        _gemm_body,
        grid_spec=grid_spec,
        out_shape=jax.ShapeDtypeStruct((P, H2), jnp.int32),
        compiler_params=pltpu.CompilerParams(
            dimension_semantics=("arbitrary",)),
    )(bexp, nrows, xmap, xs, wcol, gw, uw, dw)


_TPW = T // NW
_THALF = _TPW // 2


def _combine_body(ds_hbm, pos_hbm, out_hbm, idxv, rowsv, outv, sem):
    cid = lax.axis_index("c")
    sid = lax.axis_index("s")
    wid = sid * NC + cid
    tbase = wid * _TPW
    pltpu.sync_copy(pos_hbm.at[pl.ds(tbase * K, _TPW * K)], idxv)
    pltpu.async_copy(ds_hbm.at[idxv], rowsv, sem).wait()

    for half in range(2):
        def _addtok(tt, _):
            r = (half * _THALF + tt) * 2
            for v in range(H2 // L):
                sl = pl.ds(v * L, L)
                wa = rowsv[r, sl]
                wb = rowsv[r + 1, sl]
                lo = (plsc.bitcast(lax.shift_left(wa, 16), jnp.float32)
                      + plsc.bitcast(lax.shift_left(wb, 16), jnp.float32))
                hi = (plsc.bitcast(jnp.bitwise_and(wa, _MASK_HI), jnp.float32)
                      + plsc.bitcast(jnp.bitwise_and(wb, _MASK_HI),
                                     jnp.float32))
                outv[tt, sl] = lo
                outv[tt, pl.ds(H2 + v * L, L)] = hi
            return 0

        lax.fori_loop(0, _THALF, _addtok, 0)
        pltpu.sync_copy(
            outv, out_hbm.at[pl.ds(tbase + half * _THALF, _THALF)])


def _run_combine(ds, pos_flat):
    return pl.kernel(
        _combine_body,
        out_type=jax.ShapeDtypeStruct((T, H), jnp.float32),
        mesh=_mesh(),
        scratch_types=[
            pltpu.VMEM((_TPW * K,), jnp.int32),
            pltpu.VMEM((_TPW * K, H2), jnp.int32),
            pltpu.VMEM((_THALF, H), jnp.float32),
            pltpu.SemaphoreType.DMA,
        ],
        compiler_params=_SC_PARAMS,
    )(ds, pos_flat)


def kernel(hidden_states, gate_weight, gate_proj_w, up_proj_w, down_proj_w):
    b, s, h = hidden_states.shape
    x = hidden_states.reshape(-1, h)

    pos, wts, xi, bexp, nrows, xmap = _run_router(x, gate_weight)
    pos_flat = pos.reshape(-1)
    xs, wsort = _run_dispatch(xi, pos_flat, wts.reshape(-1))
    ds = _run_gemm(bexp.reshape(-1), nrows.reshape(-1), xmap.reshape(-1),
                   xs, wsort.reshape(P, 1),
                   gate_proj_w, up_proj_w, down_proj_w)
    out = _run_combine(ds, pos_flat)
    return out.reshape(b, s, h)


# speedup vs baseline: 1.6513x; 1.6495x over previous
import functools

import jax
import jax.numpy as jnp
from jax import lax
from jax.experimental import pallas as pl
from jax.experimental.pallas import tpu as pltpu
from jax.experimental.pallas import tpu_sc as plsc

E = 8
K = 2
T = 2048
H = 1024
F = 768
H2 = H // 2
BLK = 256
P = T * K + E * BLK
NB = P // BLK
BT = 256
NBR = T // BT

NC, NS, L = 2, 16, 16
NW = NC * NS
PAIRS = T * K

_SC_PARAMS = pltpu.CompilerParams(needs_layout_passes=False)

_mesh = functools.partial(
    plsc.VectorSubcoreMesh, core_axis_name="c", subcore_axis_name="s")

_MASK_HI = -65536


def _pack_rows(y):
    lo = lax.bitcast_convert_type(
        y[:, :H2].astype(jnp.bfloat16).astype(jnp.float32), jnp.int32)
    hi = lax.bitcast_convert_type(
        y[:, H2:].astype(jnp.bfloat16).astype(jnp.float32), jnp.int32)
    return jnp.bitwise_or(lax.shift_right_logical(lo, 16),
                          jnp.bitwise_and(hi, _MASK_HI))


def _unpack_rows(xi):
    lo = lax.bitcast_convert_type(lax.shift_left(xi, 16), jnp.float32)
    hi = lax.bitcast_convert_type(jnp.bitwise_and(xi, _MASK_HI), jnp.float32)
    return jnp.concatenate([lo, hi], axis=1)


def _top2(x, gw):
    logits = lax.dot_general(x, gw, (((1,), (1,)), ((), ())),
                             preferred_element_type=jnp.float32)
    s = jax.nn.softmax(logits, axis=-1)
    lanes = lax.broadcasted_iota(jnp.int32, s.shape, 1)
    m1 = jnp.max(s, axis=-1, keepdims=True)
    i1 = jnp.argmax(s, axis=-1)[:, None]
    s2 = jnp.where(lanes == i1, -jnp.inf, s)
    m2 = jnp.max(s2, axis=-1, keepdims=True)
    i2 = jnp.argmax(s2, axis=-1)[:, None]
    denom = m1 + m2
    oh1 = (lanes == i1).astype(jnp.float32)
    oh2 = (lanes == i2).astype(jnp.float32)
    return oh1, oh2, m1 / denom, m2 / denom


def _router_body(x_ref, gw_ref, pos_ref, wts_ref, xi_ref, bexp_ref, nrows_ref,
                 xmap_ref, cnt_ref):
    p = pl.program_id(0)
    i = pl.program_id(1)
    x = x_ref[...]
    oh1, oh2, w1, w2 = _top2(x, gw_ref[...])
    ohsum = oh1 + oh2

    @pl.when(p == 0)
    def _pass0():
        hist = jnp.sum(ohsum, axis=0, keepdims=True)
        rows = lax.broadcasted_iota(jnp.int32, (NBR, E), 0)
        cnt_ref[...] = jnp.where(rows == i, hist, cnt_ref[...])

    @pl.when(p == 1)
    def _pass1():
        xi_ref[...] = _pack_rows(x)
        cnt = cnt_ref[...]
        ones_row = jnp.ones((1, NBR), jnp.float32)
        counts = lax.dot_general(ones_row, cnt, (((1,), (0,)), ((), ())),
                                 preferred_element_type=jnp.float32)
        sel = (lax.broadcasted_iota(jnp.int32, (1, NBR), 1) < i
               ).astype(jnp.float32)
        prefix = lax.dot_general(sel, cnt, (((1,), (0,)), ((), ())),
                                 preferred_element_type=jnp.float32)
        nblk = jnp.floor((counts + (BLK - 1)) * (1.0 / BLK))
        tri_e = (lax.broadcasted_iota(jnp.int32, (E, E), 0)
                 < lax.broadcasted_iota(jnp.int32, (E, E), 1)
                 ).astype(jnp.float32)
        start = lax.dot_general(nblk, tri_e, (((1,), (0,)), ((), ())),
                                preferred_element_type=jnp.float32)
        pad_off = start * BLK

        tl = (lax.broadcasted_iota(jnp.int32, (BT, BT), 1)
              < lax.broadcasted_iota(jnp.int32, (BT, BT), 0)
              ).astype(jnp.float32)
        pre = lax.dot_general(tl, ohsum, (((1,), (0,)), ((), ())),
                              preferred_element_type=jnp.float32)
        base = pad_off + prefix
        pos1 = jnp.sum((pre + base) * oh1, axis=1, keepdims=True)
        pos2 = jnp.sum((pre + base) * oh2, axis=1, keepdims=True)
        pos_ref[...] = jnp.concatenate([pos1, pos2], axis=1).astype(jnp.int32)
        wts_ref[...] = jnp.concatenate([w1, w2], axis=1)

        @pl.when(i == 0)
        def _meta():
            occ = start[:, E - 1:E] + nblk[:, E - 1:E]
            nbs = lax.broadcasted_iota(jnp.int32, (1, NB), 1
                                       ).astype(jnp.float32)
            nbv = jnp.minimum(nbs, occ - 1.0)
            bexp = -jnp.ones((1, NB), jnp.float32)
            csel = jnp.zeros((1, NB), jnp.float32)
            psel = jnp.zeros((1, NB), jnp.float32)
            for e in range(E):
                st_e = start[:, e:e + 1]
                bexp = bexp + (st_e <= nbv).astype(jnp.float32)
            for e in range(E):
                is_e = (bexp == e).astype(jnp.float32)
                csel = csel + is_e * counts[:, e:e + 1]
                psel = psel + is_e * pad_off[:, e:e + 1]
            nrows = jnp.clip(csel - (nbv * BLK - psel), 0.0, float(BLK))
            nrows = jnp.where(nbs < occ, nrows, 0.0)
            bexp_ref[...] = bexp.astype(jnp.int32)
            nrows_ref[...] = nrows.astype(jnp.int32)
            xmap_ref[...] = nbv.astype(jnp.int32)


def _run_router(x, gate_weight):
    return pl.pallas_call(
        _router_body,
        grid=(2, NBR),
        in_specs=[
            pl.BlockSpec((BT, H), lambda p, i: (i, 0)),
            pl.BlockSpec((E, H), lambda p, i: (0, 0)),
        ],
        out_specs=[
            pl.BlockSpec((BT, K), lambda p, i: (i, 0)),
            pl.BlockSpec((BT, K), lambda p, i: (i, 0)),
            pl.BlockSpec((BT, H2), lambda p, i: (i, 0)),
            pl.BlockSpec((1, NB), lambda p, i: (0, 0)),
            pl.BlockSpec((1, NB), lambda p, i: (0, 0)),
            pl.BlockSpec((1, NB), lambda p, i: (0, 0)),
        ],
        out_shape=[
            jax.ShapeDtypeStruct((T, K), jnp.int32),
            jax.ShapeDtypeStruct((T, K), jnp.float32),
            jax.ShapeDtypeStruct((T, H2), jnp.int32),
            jax.ShapeDtypeStruct((1, NB), jnp.int32),
            jax.ShapeDtypeStruct((1, NB), jnp.int32),
            jax.ShapeDtypeStruct((1, NB), jnp.int32),
        ],
        scratch_shapes=[pltpu.VMEM((NBR, E), jnp.float32)],
        compiler_params=pltpu.CompilerParams(
            dimension_semantics=("arbitrary", "arbitrary")),
    )(x, gate_weight)


_RPW = P // NW


def _dispatch_body(xi_hbm, pos_hbm, w_hbm, xs_hbm, wsort_hbm,
                   posv, wv, tokv, pairv, wsv, rowsv, sem):
    cid = lax.axis_index("c")
    sid = lax.axis_index("s")
    wid = sid * NC + cid
    base = wid * _RPW

    pltpu.sync_copy(pos_hbm, posv)
    pltpu.sync_copy(w_hbm, wv)
    for c in range(_RPW // L):
        tokv[pl.ds(c * L, L)] = jnp.bitwise_and(
            base + c * L + lax.iota(jnp.int32, L), T - 1)
        pairv[pl.ds(c * L, L)] = jnp.zeros((L,), jnp.int32)

    def _scan(c, _):
        pp = posv[pl.ds(c * L, L)]
        rel = pp - base
        mask = jnp.logical_and(rel >= 0, rel < _RPW)
        rel = jnp.clip(rel, 0, _RPW - 1)
        pair = c * L + lax.iota(jnp.int32, L)
        plsc.store_scatter(pairv, [rel], pair, mask=mask)
        plsc.store_scatter(tokv, [rel],
                           lax.shift_right_logical(pair, 1), mask=mask)
        return 0

    lax.fori_loop(0, PAIRS // L, _scan, 0)

    def _wsel(c, _):
        pair = pairv[pl.ds(c * L, L)]
        wsv[pl.ds(c * L, L)] = plsc.load_gather(wv, [pair])
        return 0

    lax.fori_loop(0, _RPW // L, _wsel, 0)
    pltpu.sync_copy(wsv, wsort_hbm.at[pl.ds(base, _RPW)])
    half = _RPW // 2
    d0 = pltpu.async_copy(
        xi_hbm.at[tokv.at[pl.ds(0, half)]], rowsv.at[pl.ds(0, half)], sem)
    d1 = pltpu.async_copy(
        xi_hbm.at[tokv.at[pl.ds(half, half)]], rowsv.at[pl.ds(half, half)],
        sem)
    d0.wait()
    d1.wait()
    pltpu.sync_copy(rowsv, xs_hbm.at[pl.ds(base, _RPW)])


def _run_dispatch(xi, pos_flat, w_flat):
    return pl.kernel(
        _dispatch_body,
        out_type=[
            jax.ShapeDtypeStruct((P, H2), jnp.int32),
            jax.ShapeDtypeStruct((P,), jnp.float32),
        ],
        mesh=_mesh(),
        scratch_types=[
            pltpu.VMEM((PAIRS,), jnp.int32),
            pltpu.VMEM((PAIRS,), jnp.float32),
            pltpu.VMEM((_RPW,), jnp.int32),
            pltpu.VMEM((_RPW,), jnp.int32),
            pltpu.VMEM((_RPW,), jnp.float32),
            pltpu.VMEM((_RPW, H2), jnp.int32),
            pltpu.SemaphoreType.DMA,
        ],
        compiler_params=_SC_PARAMS,
    )(xi, pos_flat, w_flat)


def _gemm_body(bexp_ref, nrows_ref, xmap_ref, xs_ref, wcol_ref,
               gw_ref, uw_ref, dw_ref, out_ref):
    i = pl.program_id(0)

    @pl.when(nrows_ref[i] > 0)
    def _():
        x = _unpack_rows(xs_ref[...])
        g = lax.dot_general(x, gw_ref[0], (((1,), (0,)), ((), ())),
                            preferred_element_type=jnp.float32)
        u = lax.dot_general(x, uw_ref[0], (((1,), (0,)), ((), ())),
                            preferred_element_type=jnp.float32)
        hh = g * jax.nn.sigmoid(g) * u
        d = lax.dot_general(hh, dw_ref[0], (((1,), (0,)), ((), ())),
                            preferred_element_type=jnp.float32)
        out_ref[...] = _pack_rows(wcol_ref[...] * d)


def _run_gemm(bexp, nrows, xmap, xs, wcol, gw, uw, dw):
    grid_spec = pltpu.PrefetchScalarGridSpec(
        num_scalar_prefetch=3,
        grid=(NB,),
        in_specs=[
            pl.BlockSpec((BLK, H2), lambda i, be, nr, xm: (xm[i], 0)),
            pl.BlockSpec((BLK, 1), lambda i, be, nr, xm: (xm[i], 0)),
            pl.BlockSpec((1, H, F), lambda i, be, nr, xm: (be[i], 0, 0)),
            pl.BlockSpec((1, H, F), lambda i, be, nr, xm: (be[i], 0, 0)),
            pl.BlockSpec((1, F, H), lambda i, be, nr, xm: (be[i], 0, 0)),
        ],
        out_specs=pl.BlockSpec((BLK, H2), lambda i, be, nr, xm: (xm[i], 0)),
    )
    return pl.pallas_call(
        _gemm_body,
        grid_spec=grid_spec,
        out_shape=jax.ShapeDtypeStruct((P, H2), jnp.int32),
        compiler_params=pltpu.CompilerParams(
            dimension_semantics=("arbitrary",)),
    )(bexp, nrows, xmap, xs, wcol, gw, uw, dw)


_TPW = T // NW
_THALF = _TPW // 2


def _combine_body(ds_hbm, pos_hbm, out_hbm, idxv, rowsv, outv, sem):
    cid = lax.axis_index("c")
    sid = lax.axis_index("s")
    wid = sid * NC + cid
    tbase = wid * _TPW
    pltpu.sync_copy(pos_hbm.at[pl.ds(tbase * K, _TPW * K)], idxv)
    pltpu.async_copy(ds_hbm.at[idxv], rowsv, sem).wait()

    for half in range(2):
        def _addtok(tt, _):
            r = (half * _THALF + tt) * 2
            for v in range(H2 // L):
                sl = pl.ds(v * L, L)
                wa = rowsv[r, sl]
                wb = rowsv[r + 1, sl]
                lo = (plsc.bitcast(lax.shift_left(wa, 16), jnp.float32)
                      + plsc.bitcast(lax.shift_left(wb, 16), jnp.float32))
                hi = (plsc.bitcast(jnp.bitwise_and(wa, _MASK_HI), jnp.float32)
                      + plsc.bitcast(jnp.bitwise_and(wb, _MASK_HI),
                                     jnp.float32))
                outv[tt, sl] = lo
                outv[tt, pl.ds(H2 + v * L, L)] = hi
            return 0

        lax.fori_loop(0, _THALF, _addtok, 0)
        pltpu.sync_copy(
            outv, out_hbm.at[pl.ds(tbase + half * _THALF, _THALF)])


def _run_combine(ds, pos_flat):
    return pl.kernel(
        _combine_body,
        out_type=jax.ShapeDtypeStruct((T, H), jnp.float32),
        mesh=_mesh(),
        scratch_types=[
            pltpu.VMEM((_TPW * K,), jnp.int32),
            pltpu.VMEM((_TPW * K, H2), jnp.int32),
            pltpu.VMEM((_THALF, H), jnp.float32),
            pltpu.SemaphoreType.DMA,
        ],
        compiler_params=_SC_PARAMS,
    )(ds, pos_flat)


def kernel(hidden_states, gate_weight, gate_proj_w, up_proj_w, down_proj_w):
    b, s, h = hidden_states.shape
    x = hidden_states.reshape(-1, h)

    pos, wts, xi, bexp, nrows, xmap = _run_router(x, gate_weight)
    pos_flat = pos.reshape(-1)
    xs, wsort = _run_dispatch(xi, pos_flat, wts.reshape(-1))
    ds = _run_gemm(bexp.reshape(-1), nrows.reshape(-1), xmap.reshape(-1),
                   xs, wsort.reshape(P, 1),
                   gate_proj_w, up_proj_w, down_proj_w)
    out = _run_combine(ds, pos_flat)
    return out.reshape(b, s, h)


# speedup vs baseline: 1.7235x; 1.0437x over previous
import functools

import jax
import jax.numpy as jnp
from jax import lax
from jax.experimental import pallas as pl
from jax.experimental.pallas import tpu as pltpu
from jax.experimental.pallas import tpu_sc as plsc

E = 8
K = 2
T = 2048
H = 1024
F = 768
H2 = H // 2
BLK = 256
P = T * K + E * BLK
NB = P // BLK
BT = 512
NBR = T // BT

NC, NS, L = 2, 16, 16
NW = NC * NS
PAIRS = T * K

_SC_PARAMS = pltpu.CompilerParams(needs_layout_passes=False)

_mesh = functools.partial(
    plsc.VectorSubcoreMesh, core_axis_name="c", subcore_axis_name="s")

_MASK_HI = -65536


def _pack_rows(y):
    lo = lax.bitcast_convert_type(
        y[:, :H2].astype(jnp.bfloat16).astype(jnp.float32), jnp.int32)
    hi = lax.bitcast_convert_type(
        y[:, H2:].astype(jnp.bfloat16).astype(jnp.float32), jnp.int32)
    return jnp.bitwise_or(lax.shift_right_logical(lo, 16),
                          jnp.bitwise_and(hi, _MASK_HI))


def _unpack_rows(xi):
    lo = lax.bitcast_convert_type(lax.shift_left(xi, 16), jnp.float32)
    hi = lax.bitcast_convert_type(jnp.bitwise_and(xi, _MASK_HI), jnp.float32)
    return jnp.concatenate([lo, hi], axis=1)


def _top2(x, gw):
    logits = lax.dot_general(x, gw, (((1,), (1,)), ((), ())),
                             preferred_element_type=jnp.float32)
    s = jax.nn.softmax(logits, axis=-1)
    lanes = lax.broadcasted_iota(jnp.int32, s.shape, 1)
    m1 = jnp.max(s, axis=-1, keepdims=True)
    i1 = jnp.argmax(s, axis=-1)[:, None]
    s2 = jnp.where(lanes == i1, -jnp.inf, s)
    m2 = jnp.max(s2, axis=-1, keepdims=True)
    i2 = jnp.argmax(s2, axis=-1)[:, None]
    denom = m1 + m2
    oh1 = (lanes == i1).astype(jnp.float32)
    oh2 = (lanes == i2).astype(jnp.float32)
    return oh1, oh2, m1 / denom, m2 / denom


def _router_body(x_ref, gw_ref, pos_ref, wts_ref, xi_ref, bexp_ref, nrows_ref,
                 xmap_ref, cnt_ref):
    p = pl.program_id(0)
    i = pl.program_id(1)
    x = x_ref[...]
    oh1, oh2, w1, w2 = _top2(x, gw_ref[...])
    ohsum = oh1 + oh2

    @pl.when(p == 0)
    def _pass0():
        hist = jnp.sum(ohsum, axis=0, keepdims=True)
        rows = lax.broadcasted_iota(jnp.int32, (NBR, E), 0)
        cnt_ref[...] = jnp.where(rows == i, hist, cnt_ref[...])

    @pl.when(p == 1)
    def _pass1():
        xi_ref[...] = _pack_rows(x)
        cnt = cnt_ref[...]
        ones_row = jnp.ones((1, NBR), jnp.float32)
        counts = lax.dot_general(ones_row, cnt, (((1,), (0,)), ((), ())),
                                 preferred_element_type=jnp.float32)
        sel = (lax.broadcasted_iota(jnp.int32, (1, NBR), 1) < i
               ).astype(jnp.float32)
        prefix = lax.dot_general(sel, cnt, (((1,), (0,)), ((), ())),
                                 preferred_element_type=jnp.float32)
        nblk = jnp.floor((counts + (BLK - 1)) * (1.0 / BLK))
        tri_e = (lax.broadcasted_iota(jnp.int32, (E, E), 0)
                 < lax.broadcasted_iota(jnp.int32, (E, E), 1)
                 ).astype(jnp.float32)
        start = lax.dot_general(nblk, tri_e, (((1,), (0,)), ((), ())),
                                preferred_element_type=jnp.float32)
        pad_off = start * BLK

        tl = (lax.broadcasted_iota(jnp.int32, (BT, BT), 1)
              < lax.broadcasted_iota(jnp.int32, (BT, BT), 0)
              ).astype(jnp.float32)
        pre = lax.dot_general(tl, ohsum, (((1,), (0,)), ((), ())),
                              preferred_element_type=jnp.float32)
        base = pad_off + prefix
        pos1 = jnp.sum((pre + base) * oh1, axis=1, keepdims=True)
        pos2 = jnp.sum((pre + base) * oh2, axis=1, keepdims=True)
        pos_ref[...] = jnp.concatenate([pos1, pos2], axis=1).astype(jnp.int32)
        wts_ref[...] = jnp.concatenate([w1, w2], axis=1)

        @pl.when(i == 0)
        def _meta():
            occ = start[:, E - 1:E] + nblk[:, E - 1:E]
            nbs = lax.broadcasted_iota(jnp.int32, (1, NB), 1
                                       ).astype(jnp.float32)
            nbv = jnp.minimum(nbs, occ - 1.0)
            bexp = -jnp.ones((1, NB), jnp.float32)
            csel = jnp.zeros((1, NB), jnp.float32)
            psel = jnp.zeros((1, NB), jnp.float32)
            for e in range(E):
                st_e = start[:, e:e + 1]
                bexp = bexp + (st_e <= nbv).astype(jnp.float32)
            for e in range(E):
                is_e = (bexp == e).astype(jnp.float32)
                csel = csel + is_e * counts[:, e:e + 1]
                psel = psel + is_e * pad_off[:, e:e + 1]
            nrows = jnp.clip(csel - (nbv * BLK - psel), 0.0, float(BLK))
            nrows = jnp.where(nbs < occ, nrows, 0.0)
            bexp_ref[...] = bexp.astype(jnp.int32)
            nrows_ref[...] = nrows.astype(jnp.int32)
            xmap_ref[...] = nbv.astype(jnp.int32)


def _run_router(x, gate_weight):
    return pl.pallas_call(
        _router_body,
        grid=(2, NBR),
        in_specs=[
            pl.BlockSpec((BT, H), lambda p, i: (i, 0)),
            pl.BlockSpec((E, H), lambda p, i: (0, 0)),
        ],
        out_specs=[
            pl.BlockSpec((BT, K), lambda p, i: (i, 0)),
            pl.BlockSpec((BT, K), lambda p, i: (i, 0)),
            pl.BlockSpec((BT, H2), lambda p, i: (i, 0)),
            pl.BlockSpec((1, NB), lambda p, i: (0, 0)),
            pl.BlockSpec((1, NB), lambda p, i: (0, 0)),
            pl.BlockSpec((1, NB), lambda p, i: (0, 0)),
        ],
        out_shape=[
            jax.ShapeDtypeStruct((T, K), jnp.int32),
            jax.ShapeDtypeStruct((T, K), jnp.float32),
            jax.ShapeDtypeStruct((T, H2), jnp.int32),
            jax.ShapeDtypeStruct((1, NB), jnp.int32),
            jax.ShapeDtypeStruct((1, NB), jnp.int32),
            jax.ShapeDtypeStruct((1, NB), jnp.int32),
        ],
        scratch_shapes=[pltpu.VMEM((NBR, E), jnp.float32)],
        compiler_params=pltpu.CompilerParams(
            dimension_semantics=("arbitrary", "arbitrary")),
    )(x, gate_weight)


_RPW = P // NW


def _dispatch_body(xi_hbm, pos_hbm, w_hbm, xs_hbm, wsort_hbm,
                   posv, wv, tokv, pairv, wsv, rowsv, sem):
    cid = lax.axis_index("c")
    sid = lax.axis_index("s")
    wid = sid * NC + cid
    base = wid * _RPW

    pltpu.sync_copy(pos_hbm, posv)
    pltpu.sync_copy(w_hbm, wv)
    for c in range(_RPW // L):
        tokv[pl.ds(c * L, L)] = jnp.bitwise_and(
            base + c * L + lax.iota(jnp.int32, L), T - 1)
        pairv[pl.ds(c * L, L)] = jnp.zeros((L,), jnp.int32)

    def _scan(c, _):
        pp = posv[pl.ds(c * L, L)]
        rel = pp - base
        mask = jnp.logical_and(rel >= 0, rel < _RPW)
        rel = jnp.clip(rel, 0, _RPW - 1)
        pair = c * L + lax.iota(jnp.int32, L)
        plsc.store_scatter(pairv, [rel], pair, mask=mask)
        plsc.store_scatter(tokv, [rel],
                           lax.shift_right_logical(pair, 1), mask=mask)
        return 0

    lax.fori_loop(0, PAIRS // L, _scan, 0)

    def _wsel(c, _):
        pair = pairv[pl.ds(c * L, L)]
        wsv[pl.ds(c * L, L)] = plsc.load_gather(wv, [pair])
        return 0

    lax.fori_loop(0, _RPW // L, _wsel, 0)
    pltpu.sync_copy(wsv, wsort_hbm.at[pl.ds(base, _RPW)])
    half = _RPW // 2
    d0 = pltpu.async_copy(
        xi_hbm.at[tokv.at[pl.ds(0, half)]], rowsv.at[pl.ds(0, half)], sem)
    d1 = pltpu.async_copy(
        xi_hbm.at[tokv.at[pl.ds(half, half)]], rowsv.at[pl.ds(half, half)],
        sem)
    d0.wait()
    d1.wait()
    pltpu.sync_copy(rowsv, xs_hbm.at[pl.ds(base, _RPW)])


def _run_dispatch(xi, pos_flat, w_flat):
    return pl.kernel(
        _dispatch_body,
        out_type=[
            jax.ShapeDtypeStruct((P, H2), jnp.int32),
            jax.ShapeDtypeStruct((P,), jnp.float32),
        ],
        mesh=_mesh(),
        scratch_types=[
            pltpu.VMEM((PAIRS,), jnp.int32),
            pltpu.VMEM((PAIRS,), jnp.float32),
            pltpu.VMEM((_RPW,), jnp.int32),
            pltpu.VMEM((_RPW,), jnp.int32),
            pltpu.VMEM((_RPW,), jnp.float32),
            pltpu.VMEM((_RPW, H2), jnp.int32),
            pltpu.SemaphoreType.DMA,
        ],
        compiler_params=_SC_PARAMS,
    )(xi, pos_flat, w_flat)


def _gemm_body(bexp_ref, nrows_ref, xmap_ref, xs_ref, wcol_ref,
               gw_ref, uw_ref, dw_ref, out_ref):
    i = pl.program_id(0)

    @pl.when(nrows_ref[i] > 0)
    def _():
        x = _unpack_rows(xs_ref[...])
        g = lax.dot_general(x, gw_ref[0], (((1,), (0,)), ((), ())),
                            preferred_element_type=jnp.float32)
        u = lax.dot_general(x, uw_ref[0], (((1,), (0,)), ((), ())),
                            preferred_element_type=jnp.float32)
        hh = g * jax.nn.sigmoid(g) * u
        d = lax.dot_general(hh, dw_ref[0], (((1,), (0,)), ((), ())),
                            preferred_element_type=jnp.float32)
        out_ref[...] = _pack_rows(wcol_ref[...] * d)


def _run_gemm(bexp, nrows, xmap, xs, wcol, gw, uw, dw):
    grid_spec = pltpu.PrefetchScalarGridSpec(
        num_scalar_prefetch=3,
        grid=(NB,),
        in_specs=[
            pl.BlockSpec((BLK, H2), lambda i, be, nr, xm: (xm[i], 0)),
            pl.BlockSpec((BLK, 1), lambda i, be, nr, xm: (xm[i], 0)),
            pl.BlockSpec((1, H, F), lambda i, be, nr, xm: (be[i], 0, 0)),
            pl.BlockSpec((1, H, F), lambda i, be, nr, xm: (be[i], 0, 0)),
            pl.BlockSpec((1, F, H), lambda i, be, nr, xm: (be[i], 0, 0)),
        ],
        out_specs=pl.BlockSpec((BLK, H2), lambda i, be, nr, xm: (xm[i], 0)),
    )
    return pl.pallas_call(
        _gemm_body,
        grid_spec=grid_spec,
        out_shape=jax.ShapeDtypeStruct((P, H2), jnp.int32),
        compiler_params=pltpu.CompilerParams(
            dimension_semantics=("arbitrary",)),
    )(bexp, nrows, xmap, xs, wcol, gw, uw, dw)


_TPW = T // NW
_THALF = _TPW // 2


def _combine_body(ds_hbm, pos_hbm, out_hbm, idxv, rowsv, outv, sem):
    cid = lax.axis_index("c")
    sid = lax.axis_index("s")
    wid = sid * NC + cid
    tbase = wid * _TPW
    pltpu.sync_copy(pos_hbm.at[pl.ds(tbase * K, _TPW * K)], idxv)
    pltpu.async_copy(ds_hbm.at[idxv], rowsv, sem).wait()

    for half in range(2):
        def _addtok(tt, _):
            r = (half * _THALF + tt) * 2
            for v in range(H2 // L):
                sl = pl.ds(v * L, L)
                wa = rowsv[r, sl]
                wb = rowsv[r + 1, sl]
                lo = (plsc.bitcast(lax.shift_left(wa, 16), jnp.float32)
                      + plsc.bitcast(lax.shift_left(wb, 16), jnp.float32))
                hi = (plsc.bitcast(jnp.bitwise_and(wa, _MASK_HI), jnp.float32)
                      + plsc.bitcast(jnp.bitwise_and(wb, _MASK_HI),
                                     jnp.float32))
                outv[tt, sl] = lo
                outv[tt, pl.ds(H2 + v * L, L)] = hi
            return 0

        lax.fori_loop(0, _THALF, _addtok, 0)
        pltpu.sync_copy(
            outv, out_hbm.at[pl.ds(tbase + half * _THALF, _THALF)])


def _run_combine(ds, pos_flat):
    return pl.kernel(
        _combine_body,
        out_type=jax.ShapeDtypeStruct((T, H), jnp.float32),
        mesh=_mesh(),
        scratch_types=[
            pltpu.VMEM((_TPW * K,), jnp.int32),
            pltpu.VMEM((_TPW * K, H2), jnp.int32),
            pltpu.VMEM((_THALF, H), jnp.float32),
            pltpu.SemaphoreType.DMA,
        ],
        compiler_params=_SC_PARAMS,
    )(ds, pos_flat)


def kernel(hidden_states, gate_weight, gate_proj_w, up_proj_w, down_proj_w):
    b, s, h = hidden_states.shape
    x = hidden_states.reshape(-1, h)

    pos, wts, xi, bexp, nrows, xmap = _run_router(x, gate_weight)
    pos_flat = pos.reshape(-1)
    xs, wsort = _run_dispatch(xi, pos_flat, wts.reshape(-1))
    ds = _run_gemm(bexp.reshape(-1), nrows.reshape(-1), xmap.reshape(-1),
                   xs, wsort.reshape(P, 1),
                   gate_proj_w, up_proj_w, down_proj_w)
    out = _run_combine(ds, pos_flat)
    return out.reshape(b, s, h)


# speedup vs baseline: 1.7589x; 1.0205x over previous
import functools

import jax
import jax.numpy as jnp
from jax import lax
from jax.experimental import pallas as pl
from jax.experimental.pallas import tpu as pltpu
from jax.experimental.pallas import tpu_sc as plsc

E = 8
K = 2
T = 2048
H = 1024
F = 768
H2 = H // 2
BLK = 256
P = T * K + E * BLK
NB = P // BLK
BT = 1024
NBR = T // BT

NC, NS, L = 2, 16, 16
NW = NC * NS
PAIRS = T * K

_SC_PARAMS = pltpu.CompilerParams(needs_layout_passes=False)

_mesh = functools.partial(
    plsc.VectorSubcoreMesh, core_axis_name="c", subcore_axis_name="s")

_MASK_HI = -65536


def _pack_rows(y):
    lo = lax.bitcast_convert_type(
        y[:, :H2].astype(jnp.bfloat16).astype(jnp.float32), jnp.int32)
    hi = lax.bitcast_convert_type(
        y[:, H2:].astype(jnp.bfloat16).astype(jnp.float32), jnp.int32)
    return jnp.bitwise_or(lax.shift_right_logical(lo, 16),
                          jnp.bitwise_and(hi, _MASK_HI))


def _unpack_rows(xi):
    lo = lax.bitcast_convert_type(lax.shift_left(xi, 16), jnp.float32)
    hi = lax.bitcast_convert_type(jnp.bitwise_and(xi, _MASK_HI), jnp.float32)
    return jnp.concatenate([lo, hi], axis=1)


def _top2(x, gw):
    logits = lax.dot_general(x, gw, (((1,), (1,)), ((), ())),
                             preferred_element_type=jnp.float32)
    s = jax.nn.softmax(logits, axis=-1)
    lanes = lax.broadcasted_iota(jnp.int32, s.shape, 1)
    m1 = jnp.max(s, axis=-1, keepdims=True)
    i1 = jnp.argmax(s, axis=-1)[:, None]
    s2 = jnp.where(lanes == i1, -jnp.inf, s)
    m2 = jnp.max(s2, axis=-1, keepdims=True)
    i2 = jnp.argmax(s2, axis=-1)[:, None]
    denom = m1 + m2
    oh1 = (lanes == i1).astype(jnp.float32)
    oh2 = (lanes == i2).astype(jnp.float32)
    return oh1, oh2, m1 / denom, m2 / denom


def _router_body(x_ref, gw_ref, pos_ref, wts_ref, xi_ref, bexp_ref, nrows_ref,
                 xmap_ref, cnt_ref):
    p = pl.program_id(0)
    i = pl.program_id(1)
    x = x_ref[...]
    oh1, oh2, w1, w2 = _top2(x, gw_ref[...])
    ohsum = oh1 + oh2

    @pl.when(p == 0)
    def _pass0():
        hist = jnp.sum(ohsum, axis=0, keepdims=True)
        rows = lax.broadcasted_iota(jnp.int32, (NBR, E), 0)
        cnt_ref[...] = jnp.where(rows == i, hist, cnt_ref[...])

    @pl.when(p == 1)
    def _pass1():
        xi_ref[...] = _pack_rows(x)
        cnt = cnt_ref[...]
        ones_row = jnp.ones((1, NBR), jnp.float32)
        counts = lax.dot_general(ones_row, cnt, (((1,), (0,)), ((), ())),
                                 preferred_element_type=jnp.float32)
        sel = (lax.broadcasted_iota(jnp.int32, (1, NBR), 1) < i
               ).astype(jnp.float32)
        prefix = lax.dot_general(sel, cnt, (((1,), (0,)), ((), ())),
                                 preferred_element_type=jnp.float32)
        nblk = jnp.floor((counts + (BLK - 1)) * (1.0 / BLK))
        tri_e = (lax.broadcasted_iota(jnp.int32, (E, E), 0)
                 < lax.broadcasted_iota(jnp.int32, (E, E), 1)
                 ).astype(jnp.float32)
        start = lax.dot_general(nblk, tri_e, (((1,), (0,)), ((), ())),
                                preferred_element_type=jnp.float32)
        pad_off = start * BLK

        tl = (lax.broadcasted_iota(jnp.int32, (BT, BT), 1)
              < lax.broadcasted_iota(jnp.int32, (BT, BT), 0)
              ).astype(jnp.float32)
        pre = lax.dot_general(tl, ohsum, (((1,), (0,)), ((), ())),
                              preferred_element_type=jnp.float32)
        base = pad_off + prefix
        pos1 = jnp.sum((pre + base) * oh1, axis=1, keepdims=True)
        pos2 = jnp.sum((pre + base) * oh2, axis=1, keepdims=True)
        pos_ref[...] = jnp.concatenate([pos1, pos2], axis=1).astype(jnp.int32)
        wts_ref[...] = jnp.concatenate([w1, w2], axis=1)

        @pl.when(i == 0)
        def _meta():
            occ = start[:, E - 1:E] + nblk[:, E - 1:E]
            nbs = lax.broadcasted_iota(jnp.int32, (1, NB), 1
                                       ).astype(jnp.float32)
            nbv = jnp.minimum(nbs, occ - 1.0)
            bexp = -jnp.ones((1, NB), jnp.float32)
            csel = jnp.zeros((1, NB), jnp.float32)
            psel = jnp.zeros((1, NB), jnp.float32)
            for e in range(E):
                st_e = start[:, e:e + 1]
                bexp = bexp + (st_e <= nbv).astype(jnp.float32)
            for e in range(E):
                is_e = (bexp == e).astype(jnp.float32)
                csel = csel + is_e * counts[:, e:e + 1]
                psel = psel + is_e * pad_off[:, e:e + 1]
            nrows = jnp.clip(csel - (nbv * BLK - psel), 0.0, float(BLK))
            nrows = jnp.where(nbs < occ, nrows, 0.0)
            bexp_ref[...] = bexp.astype(jnp.int32)
            nrows_ref[...] = nrows.astype(jnp.int32)
            xmap_ref[...] = nbv.astype(jnp.int32)


def _run_router(x, gate_weight):
    return pl.pallas_call(
        _router_body,
        grid=(2, NBR),
        in_specs=[
            pl.BlockSpec((BT, H), lambda p, i: (i, 0)),
            pl.BlockSpec((E, H), lambda p, i: (0, 0)),
        ],
        out_specs=[
            pl.BlockSpec((BT, K), lambda p, i: (i, 0)),
            pl.BlockSpec((BT, K), lambda p, i: (i, 0)),
            pl.BlockSpec((BT, H2), lambda p, i: (i, 0)),
            pl.BlockSpec((1, NB), lambda p, i: (0, 0)),
            pl.BlockSpec((1, NB), lambda p, i: (0, 0)),
            pl.BlockSpec((1, NB), lambda p, i: (0, 0)),
        ],
        out_shape=[
            jax.ShapeDtypeStruct((T, K), jnp.int32),
            jax.ShapeDtypeStruct((T, K), jnp.float32),
            jax.ShapeDtypeStruct((T, H2), jnp.int32),
            jax.ShapeDtypeStruct((1, NB), jnp.int32),
            jax.ShapeDtypeStruct((1, NB), jnp.int32),
            jax.ShapeDtypeStruct((1, NB), jnp.int32),
        ],
        scratch_shapes=[pltpu.VMEM((NBR, E), jnp.float32)],
        compiler_params=pltpu.CompilerParams(
            dimension_semantics=("arbitrary", "arbitrary")),
    )(x, gate_weight)


_RPW = P // NW


def _dispatch_body(xi_hbm, pos_hbm, w_hbm, xs_hbm, wsort_hbm,
                   posv, wv, tokv, pairv, wsv, rowsv, sem):
    cid = lax.axis_index("c")
    sid = lax.axis_index("s")
    wid = sid * NC + cid
    base = wid * _RPW

    pltpu.sync_copy(pos_hbm, posv)
    pltpu.sync_copy(w_hbm, wv)
    for c in range(_RPW // L):
        tokv[pl.ds(c * L, L)] = jnp.bitwise_and(
            base + c * L + lax.iota(jnp.int32, L), T - 1)
        pairv[pl.ds(c * L, L)] = jnp.zeros((L,), jnp.int32)

    def _scan(c, _):
        pp = posv[pl.ds(c * L, L)]
        rel = pp - base
        mask = jnp.logical_and(rel >= 0, rel < _RPW)
        rel = jnp.clip(rel, 0, _RPW - 1)
        pair = c * L + lax.iota(jnp.int32, L)
        plsc.store_scatter(pairv, [rel], pair, mask=mask)
        plsc.store_scatter(tokv, [rel],
                           lax.shift_right_logical(pair, 1), mask=mask)
        return 0

    lax.fori_loop(0, PAIRS // L, _scan, 0)

    def _wsel(c, _):
        pair = pairv[pl.ds(c * L, L)]
        wsv[pl.ds(c * L, L)] = plsc.load_gather(wv, [pair])
        return 0

    lax.fori_loop(0, _RPW // L, _wsel, 0)
    pltpu.sync_copy(wsv, wsort_hbm.at[pl.ds(base, _RPW)])
    half = _RPW // 2
    d0 = pltpu.async_copy(
        xi_hbm.at[tokv.at[pl.ds(0, half)]], rowsv.at[pl.ds(0, half)], sem)
    d1 = pltpu.async_copy(
        xi_hbm.at[tokv.at[pl.ds(half, half)]], rowsv.at[pl.ds(half, half)],
        sem)
    d0.wait()
    d1.wait()
    pltpu.sync_copy(rowsv, xs_hbm.at[pl.ds(base, _RPW)])


def _run_dispatch(xi, pos_flat, w_flat):
    return pl.kernel(
        _dispatch_body,
        out_type=[
            jax.ShapeDtypeStruct((P, H2), jnp.int32),
            jax.ShapeDtypeStruct((P,), jnp.float32),
        ],
        mesh=_mesh(),
        scratch_types=[
            pltpu.VMEM((PAIRS,), jnp.int32),
            pltpu.VMEM((PAIRS,), jnp.float32),
            pltpu.VMEM((_RPW,), jnp.int32),
            pltpu.VMEM((_RPW,), jnp.int32),
            pltpu.VMEM((_RPW,), jnp.float32),
            pltpu.VMEM((_RPW, H2), jnp.int32),
            pltpu.SemaphoreType.DMA,
        ],
        compiler_params=_SC_PARAMS,
    )(xi, pos_flat, w_flat)


def _gemm_body(bexp_ref, nrows_ref, xmap_ref, xs_ref, wcol_ref,
               gw_ref, uw_ref, dw_ref, out_ref):
    i = pl.program_id(0)

    @pl.when(nrows_ref[i] > 0)
    def _():
        x = _unpack_rows(xs_ref[...])
        g = lax.dot_general(x, gw_ref[0], (((1,), (0,)), ((), ())),
                            preferred_element_type=jnp.float32)
        u = lax.dot_general(x, uw_ref[0], (((1,), (0,)), ((), ())),
                            preferred_element_type=jnp.float32)
        hh = g * jax.nn.sigmoid(g) * u
        d = lax.dot_general(hh, dw_ref[0], (((1,), (0,)), ((), ())),
                            preferred_element_type=jnp.float32)
        out_ref[...] = _pack_rows(wcol_ref[...] * d)


def _run_gemm(bexp, nrows, xmap, xs, wcol, gw, uw, dw):
    grid_spec = pltpu.PrefetchScalarGridSpec(
        num_scalar_prefetch=3,
        grid=(NB,),
        in_specs=[
            pl.BlockSpec((BLK, H2), lambda i, be, nr, xm: (xm[i], 0)),
            pl.BlockSpec((BLK, 1), lambda i, be, nr, xm: (xm[i], 0)),
            pl.BlockSpec((1, H, F), lambda i, be, nr, xm: (be[i], 0, 0)),
            pl.BlockSpec((1, H, F), lambda i, be, nr, xm: (be[i], 0, 0)),
            pl.BlockSpec((1, F, H), lambda i, be, nr, xm: (be[i], 0, 0)),
        ],
        out_specs=pl.BlockSpec((BLK, H2), lambda i, be, nr, xm: (xm[i], 0)),
    )
    return pl.pallas_call(
        _gemm_body,
        grid_spec=grid_spec,
        out_shape=jax.ShapeDtypeStruct((P, H2), jnp.int32),
        compiler_params=pltpu.CompilerParams(
            dimension_semantics=("arbitrary",)),
    )(bexp, nrows, xmap, xs, wcol, gw, uw, dw)


_TPW = T // NW
_THALF = _TPW // 2


def _combine_body(ds_hbm, pos_hbm, out_hbm, idxv, rowsv, outv, sem):
    cid = lax.axis_index("c")
    sid = lax.axis_index("s")
    wid = sid * NC + cid
    tbase = wid * _TPW
    pltpu.sync_copy(pos_hbm.at[pl.ds(tbase * K, _TPW * K)], idxv)
    pltpu.async_copy(ds_hbm.at[idxv], rowsv, sem).wait()

    for half in range(2):
        def _addtok(tt, _):
            r = (half * _THALF + tt) * 2
            for v in range(H2 // L):
                sl = pl.ds(v * L, L)
                wa = rowsv[r, sl]
                wb = rowsv[r + 1, sl]
                lo = (plsc.bitcast(lax.shift_left(wa, 16), jnp.float32)
                      + plsc.bitcast(lax.shift_left(wb, 16), jnp.float32))
                hi = (plsc.bitcast(jnp.bitwise_and(wa, _MASK_HI), jnp.float32)
                      + plsc.bitcast(jnp.bitwise_and(wb, _MASK_HI),
                                     jnp.float32))
                outv[tt, sl] = lo
                outv[tt, pl.ds(H2 + v * L, L)] = hi
            return 0

        lax.fori_loop(0, _THALF, _addtok, 0)
        pltpu.sync_copy(
            outv, out_hbm.at[pl.ds(tbase + half * _THALF, _THALF)])


def _run_combine(ds, pos_flat):
    return pl.kernel(
        _combine_body,
        out_type=jax.ShapeDtypeStruct((T, H), jnp.float32),
        mesh=_mesh(),
        scratch_types=[
            pltpu.VMEM((_TPW * K,), jnp.int32),
            pltpu.VMEM((_TPW * K, H2), jnp.int32),
            pltpu.VMEM((_THALF, H), jnp.float32),
            pltpu.SemaphoreType.DMA,
        ],
        compiler_params=_SC_PARAMS,
    )(ds, pos_flat)


def kernel(hidden_states, gate_weight, gate_proj_w, up_proj_w, down_proj_w):
    b, s, h = hidden_states.shape
    x = hidden_states.reshape(-1, h)

    pos, wts, xi, bexp, nrows, xmap = _run_router(x, gate_weight)
    pos_flat = pos.reshape(-1)
    xs, wsort = _run_dispatch(xi, pos_flat, wts.reshape(-1))
    ds = _run_gemm(bexp.reshape(-1), nrows.reshape(-1), xmap.reshape(-1),
                   xs, wsort.reshape(P, 1),
                   gate_proj_w, up_proj_w, down_proj_w)
    out = _run_combine(ds, pos_flat)
    return out.reshape(b, s, h)
